# Initial kernel scaffold; baseline (speedup 1.0000x reference)
import jax
import jax.numpy as jnp
from jax.experimental import pallas as pl


def kernel(projected_patches, num_patches, pos_table):
    raise NotImplementedError("write your pallas kernel here")


# speedup vs baseline: 1.0123x; 1.0123x over previous
import jax
import jax.numpy as jnp
from jax.experimental import pallas as pl
from jax.experimental.pallas import tpu as pltpu


def _pe_kernel(np_ref, x_ref, table_ref, o_ref):
    n = np_ref[0]
    table = table_ref[...]
    last = table_ref[pl.ds(n - 1, 1), :]
    rows = jax.lax.broadcasted_iota(jnp.int32, (table.shape[0], 1), 0)
    enc = jnp.where(rows < n, table, last)
    o_ref[...] = x_ref[...] + enc[None]


def kernel(projected_patches, num_patches, pos_table):
    B, P, D = projected_patches.shape
    np_arr = jnp.asarray(num_patches, jnp.int32).reshape((1,))
    grid_spec = pltpu.PrefetchScalarGridSpec(
        num_scalar_prefetch=1,
        grid=(B,),
        in_specs=[
            pl.BlockSpec((1, P, D), lambda b, np_: (b, 0, 0)),
            pl.BlockSpec((P, D), lambda b, np_: (0, 0)),
        ],
        out_specs=pl.BlockSpec((1, P, D), lambda b, np_: (b, 0, 0)),
    )
    return pl.pallas_call(
        _pe_kernel,
        grid_spec=grid_spec,
        out_shape=jax.ShapeDtypeStruct((B, P, D), projected_patches.dtype),
    )(np_arr, projected_patches, pos_table)


# speedup vs baseline: 1.0451x; 1.0324x over previous
import jax
import jax.numpy as jnp
from jax.experimental import pallas as pl
from jax.experimental.pallas import tpu as pltpu


def _pe_kernel(np_ref, x_ref, table_ref, o_ref):
    n = np_ref[0]
    table = table_ref[...]
    last = table_ref[pl.ds(n - 1, 1), :]
    rows = jax.lax.broadcasted_iota(jnp.int32, (table.shape[0], 1), 0)
    enc = jnp.where(rows < n, table, last)
    o_ref[...] = x_ref[...] + enc[None]


def kernel(projected_patches, num_patches, pos_table):
    B, P, D = projected_patches.shape
    np_arr = jnp.asarray(num_patches, jnp.int32).reshape((1,))
    BB = 2
    grid_spec = pltpu.PrefetchScalarGridSpec(
        num_scalar_prefetch=1,
        grid=(B // BB,),
        in_specs=[
            pl.BlockSpec((BB, P, D), lambda b, np_: (b, 0, 0)),
            pl.BlockSpec((P, D), lambda b, np_: (0, 0)),
        ],
        out_specs=pl.BlockSpec((BB, P, D), lambda b, np_: (b, 0, 0)),
    )
    return pl.pallas_call(
        _pe_kernel,
        grid_spec=grid_spec,
        out_shape=jax.ShapeDtypeStruct((B, P, D), projected_patches.dtype),
    )(np_arr, projected_patches, pos_table)


# speedup vs baseline: 1.0569x; 1.0113x over previous
import jax
import jax.numpy as jnp
from jax.experimental import pallas as pl
from jax.experimental.pallas import tpu as pltpu


def _pe_kernel(np_ref, x_ref, table_ref, o_ref):
    n = np_ref[0]
    table = table_ref[...]
    last = table_ref[pl.ds(n - 1, 1), :]
    rows = jax.lax.broadcasted_iota(jnp.int32, (table.shape[0], 1), 0)
    enc = jnp.where(rows < n, table, last)
    o_ref[...] = x_ref[...] + enc[None]


def kernel(projected_patches, num_patches, pos_table):
    B, P, D = projected_patches.shape
    np_arr = jnp.asarray(num_patches, jnp.int32).reshape((1,))
    BB = 4
    grid_spec = pltpu.PrefetchScalarGridSpec(
        num_scalar_prefetch=1,
        grid=(B // BB,),
        in_specs=[
            pl.BlockSpec((BB, P, D), lambda b, np_: (b, 0, 0)),
            pl.BlockSpec((P, D), lambda b, np_: (0, 0)),
        ],
        out_specs=pl.BlockSpec((BB, P, D), lambda b, np_: (b, 0, 0)),
    )
    return pl.pallas_call(
        _pe_kernel,
        grid_spec=grid_spec,
        out_shape=jax.ShapeDtypeStruct((B, P, D), projected_patches.dtype),
    )(np_arr, projected_patches, pos_table)
